# baseline (speedup 1.0000x reference)
import functools

import jax
import jax.numpy as jnp
from jax import lax
from jax.experimental import pallas as pl
from jax.experimental.pallas import tpu as pltpu
from jax.experimental.pallas import tpu_sc as plsc

N_TOK = 8192
N_SLOTS = 512
T = 30
D = 256
D_FFN = 1024
CELLS = N_SLOTS * T

L = 16
NC = 2
NS = 16
NW = NC * NS
CPT = CELLS // NW
WIN = 96
NWIN = CPT // WIN

BLK = 256
PAD = BLK
Y_ROWS = N_TOK + PAD


def _ffn_body(x_ref, w1_ref, b1_ref, w2_ref, b2_ref, g_ref, bt_ref, y_ref):
    i = pl.program_id(0)
    x = x_ref[...]
    h = jax.nn.gelu(
        jnp.dot(x, w1_ref[...], preferred_element_type=jnp.float32) + b1_ref[...]
    )
    f = jnp.dot(h, w2_ref[...], preferred_element_type=jnp.float32) + b2_ref[...]
    xr = x + f
    mean = jnp.mean(xr, axis=-1, keepdims=True)
    var = jnp.mean((xr - mean) ** 2, axis=-1, keepdims=True)
    res = (xr - mean) * lax.rsqrt(var + 1e-5) * g_ref[...] + bt_ref[...]
    y_ref[...] = jnp.where(i < N_TOK // BLK, res, 0.0)


def _ffn(x, w1, b1, w2, b2, gamma, beta):
    nblk = Y_ROWS // BLK
    last = N_TOK // BLK - 1
    return pl.pallas_call(
        _ffn_body,
        grid=(nblk,),
        in_specs=[
            pl.BlockSpec((BLK, D), lambda i: (jnp.minimum(i, last), 0)),
            pl.BlockSpec((D, D_FFN), lambda i: (0, 0)),
            pl.BlockSpec((D_FFN,), lambda i: (0,)),
            pl.BlockSpec((D_FFN, D), lambda i: (0, 0)),
            pl.BlockSpec((D,), lambda i: (0,)),
            pl.BlockSpec((D,), lambda i: (0,)),
            pl.BlockSpec((D,), lambda i: (0,)),
        ],
        out_specs=pl.BlockSpec((BLK, D), lambda i: (i, 0)),
        out_shape=jax.ShapeDtypeStruct((Y_ROWS, D), jnp.float32),
    )(x, w1, b1, w2, b2, gamma, beta)


def _sc_route(slot_i32, frame_i32, y):
    mesh = plsc.VectorSubcoreMesh(core_axis_name="c", subcore_axis_name="s")

    @functools.partial(
        pl.kernel,
        mesh=mesh,
        out_type=jax.ShapeDtypeStruct((CELLS, D), jnp.float32),
        compiler_params=pltpu.CompilerParams(
            needs_layout_passes=False, use_tc_tiling_on_sc=True
        ),
        scratch_types=[
            pltpu.VMEM((N_TOK,), jnp.int32),
            pltpu.VMEM((N_TOK,), jnp.int32),
            pltpu.VMEM((CPT,), jnp.int32),
            pltpu.VMEM((NWIN, WIN), jnp.int32),
            pltpu.VMEM((2, WIN, D), jnp.float32),
            pltpu.SemaphoreType.DMA,
            pltpu.SemaphoreType.DMA,
        ],
    )
    def body(slot_hbm, frame_hbm, y_hbm, out_hbm,
             slot_v, frame_v, win_v, idx_v, rows_v, sem0, sem1):
        cid = lax.axis_index("c")
        sid = lax.axis_index("s")
        wid = sid * NC + cid
        base = wid * CPT
        lane = lax.iota(jnp.int32, L)

        pltpu.sync_copy(slot_hbm, slot_v)
        pltpu.sync_copy(frame_hbm, frame_v)

        zrow = N_TOK + wid * 4 + jnp.bitwise_and(lane, 3)
        for k in range(CPT // L):
            win_v[pl.ds(k * L, L)] = zrow

        def scan(j, carry):
            s = slot_v[pl.ds(j * L, L)]
            f = frame_v[pl.ds(j * L, L)]
            c = s * T + f
            key = c * L + lane
            tok = j * L + lane
            ks, vs = plsc.sort_key_val(key, tok)
            cs = lax.shift_right_logical(ks, 4)
            nxt = jnp.minimum(lane + 1, L - 1)
            cs_n = lax.gather(
                cs,
                nxt[:, None],
                lax.GatherDimensionNumbers(
                    offset_dims=(),
                    collapsed_slice_dims=(0,),
                    start_index_map=(0,),
                ),
                slice_sizes=(1,),
                mode=lax.GatherScatterMode.PROMISE_IN_BOUNDS,
            )
            keep = jnp.logical_or(cs != cs_n, lane == L - 1)
            owned = jnp.logical_and(cs >= base, cs < base + CPT)
            m = jnp.logical_and(keep, owned)
            li = jnp.clip(cs - base, 0, CPT - 1)
            plsc.store_scatter(win_v, [li], vs, mask=m)
            return carry

        lax.fori_loop(0, N_TOK // L, scan, 0)

        for k in range(CPT // L):
            r, off = (k * L) // WIN, (k * L) % WIN
            idx_v[r, pl.ds(off, L)] = win_v[pl.ds(k * L, L)]

        sems = (sem0, sem1)
        cps = [pltpu.async_copy(y_hbm.at[idx_v.at[0]], rows_v.at[0], sems[0])]
        for w in range(NWIN):
            if w + 1 < NWIN:
                cps.append(
                    pltpu.async_copy(
                        y_hbm.at[idx_v.at[w + 1]],
                        rows_v.at[(w + 1) % 2],
                        sems[(w + 1) % 2],
                    )
                )
            cps[w].wait()
            pltpu.sync_copy(
                rows_v.at[w % 2], out_hbm.at[pl.ds(base + w * WIN, WIN)]
            )

    return body(slot_i32, frame_i32, y)


def kernel(flat_features, slot_idx, frame_ids, W1, b1, W2, b2, gamma, beta):
    slot_i32 = slot_idx.astype(jnp.int32)
    frame_i32 = frame_ids.astype(jnp.int32)
    y = _ffn(flat_features, W1, b1, W2, b2, gamma, beta)
    out = _sc_route(slot_i32, frame_i32, y)
    return out.reshape(N_SLOTS, T, D)


# speedup vs baseline: 1.3493x; 1.3493x over previous
import functools

import jax
import jax.numpy as jnp
from jax import lax
from jax.experimental import pallas as pl
from jax.experimental.pallas import tpu as pltpu
from jax.experimental.pallas import tpu_sc as plsc

N_TOK = 8192
N_SLOTS = 512
T = 30
D = 256
D_FFN = 1024
CELLS = N_SLOTS * T

L = 16
NC = 2
NS = 16
NW = NC * NS
CPT = CELLS // NW
WIN = 96
NWIN = CPT // WIN

BLK = 256
PAD = BLK
Y_ROWS = N_TOK + PAD


def _ffn_body(x_ref, w1_ref, b1_ref, w2_ref, b2_ref, g_ref, bt_ref, y_ref):
    i = pl.program_id(0)
    x = x_ref[...]
    h = jax.nn.gelu(
        jnp.dot(x, w1_ref[...], preferred_element_type=jnp.float32) + b1_ref[...]
    )
    f = jnp.dot(h, w2_ref[...], preferred_element_type=jnp.float32) + b2_ref[...]
    xr = x + f
    mean = jnp.mean(xr, axis=-1, keepdims=True)
    var = jnp.mean((xr - mean) ** 2, axis=-1, keepdims=True)
    res = (xr - mean) * lax.rsqrt(var + 1e-5) * g_ref[...] + bt_ref[...]
    y_ref[...] = jnp.where(i < N_TOK // BLK, res, 0.0)


def _ffn(x, w1, b1, w2, b2, gamma, beta):
    nblk = Y_ROWS // BLK
    last = N_TOK // BLK - 1
    return pl.pallas_call(
        _ffn_body,
        grid=(nblk,),
        in_specs=[
            pl.BlockSpec((BLK, D), lambda i: (jnp.minimum(i, last), 0)),
            pl.BlockSpec((D, D_FFN), lambda i: (0, 0)),
            pl.BlockSpec((D_FFN,), lambda i: (0,)),
            pl.BlockSpec((D_FFN, D), lambda i: (0, 0)),
            pl.BlockSpec((D,), lambda i: (0,)),
            pl.BlockSpec((D,), lambda i: (0,)),
            pl.BlockSpec((D,), lambda i: (0,)),
        ],
        out_specs=pl.BlockSpec((BLK, D), lambda i: (i, 0)),
        out_shape=jax.ShapeDtypeStruct((Y_ROWS, D), jnp.float32),
    )(x, w1, b1, w2, b2, gamma, beta)


def _sc_route(slot_i32, frame_i32, y):
    mesh = plsc.VectorSubcoreMesh(core_axis_name="c", subcore_axis_name="s")

    @functools.partial(
        pl.kernel,
        mesh=mesh,
        out_type=jax.ShapeDtypeStruct((CELLS, D), jnp.float32),
        compiler_params=pltpu.CompilerParams(
            needs_layout_passes=False, use_tc_tiling_on_sc=True
        ),
        scratch_types=[
            pltpu.VMEM((N_TOK,), jnp.int32),
            pltpu.VMEM((N_TOK,), jnp.int32),
            pltpu.VMEM((CPT,), jnp.int32),
            pltpu.VMEM((NWIN, WIN), jnp.int32),
            pltpu.VMEM((2, WIN, D), jnp.float32),
            pltpu.SemaphoreType.DMA,
            pltpu.SemaphoreType.DMA,
        ],
    )
    def body(slot_hbm, frame_hbm, y_hbm, out_hbm,
             slot_v, frame_v, win_v, idx_v, rows_v, sem0, sem1):
        cid = lax.axis_index("c")
        sid = lax.axis_index("s")
        wid = sid * NC + cid
        base = wid * CPT
        lane = lax.iota(jnp.int32, L)

        pltpu.sync_copy(slot_hbm, slot_v)
        pltpu.sync_copy(frame_hbm, frame_v)

        zrow = N_TOK + wid * 4 + jnp.bitwise_and(lane, 3)
        for k in range(CPT // L):
            win_v[pl.ds(k * L, L)] = zrow

        def scan(j, carry):
            s = slot_v[pl.ds(j * L, L)]
            f = frame_v[pl.ds(j * L, L)]
            c = f * N_SLOTS + s
            key = c * L + lane
            tok = j * L + lane
            ks, vs = plsc.sort_key_val(key, tok)
            cs = lax.shift_right_logical(ks, 4)
            nxt = jnp.minimum(lane + 1, L - 1)
            cs_n = lax.gather(
                cs,
                nxt[:, None],
                lax.GatherDimensionNumbers(
                    offset_dims=(),
                    collapsed_slice_dims=(0,),
                    start_index_map=(0,),
                ),
                slice_sizes=(1,),
                mode=lax.GatherScatterMode.PROMISE_IN_BOUNDS,
            )
            keep = jnp.logical_or(cs != cs_n, lane == L - 1)
            owned = jnp.logical_and(cs >= base, cs < base + CPT)
            m = jnp.logical_and(keep, owned)
            li = jnp.clip(cs - base, 0, CPT - 1)
            plsc.store_scatter(win_v, [li], vs, mask=m)
            return carry

        lax.fori_loop(0, N_TOK // L, scan, 0)

        for k in range(CPT // L):
            r, off = (k * L) // WIN, (k * L) % WIN
            idx_v[r, pl.ds(off, L)] = win_v[pl.ds(k * L, L)]

        sems = (sem0, sem1)
        cps = [pltpu.async_copy(y_hbm.at[idx_v.at[0]], rows_v.at[0], sems[0])]
        for w in range(NWIN):
            if w + 1 < NWIN:
                cps.append(
                    pltpu.async_copy(
                        y_hbm.at[idx_v.at[w + 1]],
                        rows_v.at[(w + 1) % 2],
                        sems[(w + 1) % 2],
                    )
                )
            cps[w].wait()
            pltpu.sync_copy(
                rows_v.at[w % 2], out_hbm.at[pl.ds(base + w * WIN, WIN)]
            )

    return body(slot_i32, frame_i32, y)


def kernel(flat_features, slot_idx, frame_ids, W1, b1, W2, b2, gamma, beta):
    slot_i32 = slot_idx.astype(jnp.int32)
    frame_i32 = frame_ids.astype(jnp.int32)
    y = _ffn(flat_features, W1, b1, W2, b2, gamma, beta)
    out = _sc_route(slot_i32, frame_i32, y)
    return out.reshape(T, N_SLOTS, D).transpose(1, 0, 2)


# speedup vs baseline: 1.5179x; 1.1249x over previous
import functools

import jax
import jax.numpy as jnp
from jax import lax
from jax.experimental import pallas as pl
from jax.experimental.pallas import tpu as pltpu
from jax.experimental.pallas import tpu_sc as plsc

N_TOK = 8192
N_SLOTS = 512
T = 30
D = 256
D_FFN = 1024
CELLS = N_SLOTS * T

L = 16
NC = 2
NS = 16
NW = NC * NS
CPT = CELLS // NW
WIN = 96
NWIN = CPT // WIN

BLK = 256
PAD = BLK
Y_ROWS = N_TOK + PAD


def _ffn_body(x_ref, w1_ref, b1_ref, w2_ref, b2_ref, g_ref, bt_ref, y_ref):
    i = pl.program_id(0)
    x = x_ref[...]
    h = jax.nn.gelu(
        jnp.dot(x, w1_ref[...], preferred_element_type=jnp.float32) + b1_ref[...]
    )
    f = jnp.dot(h, w2_ref[...], preferred_element_type=jnp.float32) + b2_ref[...]
    xr = x + f
    mean = jnp.mean(xr, axis=-1, keepdims=True)
    var = jnp.mean((xr - mean) ** 2, axis=-1, keepdims=True)
    res = (xr - mean) * lax.rsqrt(var + 1e-5) * g_ref[...] + bt_ref[...]
    y_ref[...] = jnp.where(i < N_TOK // BLK, res, 0.0)


def _ffn(x, w1, b1, w2, b2, gamma, beta):
    nblk = Y_ROWS // BLK
    last = N_TOK // BLK - 1
    return pl.pallas_call(
        _ffn_body,
        grid=(nblk,),
        in_specs=[
            pl.BlockSpec((BLK, D), lambda i: (jnp.minimum(i, last), 0)),
            pl.BlockSpec((D, D_FFN), lambda i: (0, 0)),
            pl.BlockSpec((D_FFN,), lambda i: (0,)),
            pl.BlockSpec((D_FFN, D), lambda i: (0, 0)),
            pl.BlockSpec((D,), lambda i: (0,)),
            pl.BlockSpec((D,), lambda i: (0,)),
            pl.BlockSpec((D,), lambda i: (0,)),
        ],
        out_specs=pl.BlockSpec((BLK, D), lambda i: (i, 0)),
        out_shape=jax.ShapeDtypeStruct((Y_ROWS, D), jnp.float32),
    )(x, w1, b1, w2, b2, gamma, beta)


def _sc_scan(slot_i32, frame_i32):
    mesh = plsc.VectorSubcoreMesh(core_axis_name="c", subcore_axis_name="s")

    @functools.partial(
        pl.kernel,
        mesh=mesh,
        out_type=jax.ShapeDtypeStruct((NW, NWIN, WIN), jnp.int32),
        compiler_params=pltpu.CompilerParams(
            needs_layout_passes=False, use_tc_tiling_on_sc=True
        ),
        scratch_types=[
            pltpu.VMEM((N_TOK,), jnp.int32),
            pltpu.VMEM((N_TOK,), jnp.int32),
            pltpu.VMEM((CPT,), jnp.int32),
            pltpu.VMEM((NWIN, WIN), jnp.int32),
        ],
    )
    def body(slot_hbm, frame_hbm, idx_hbm, slot_v, frame_v, win_v, idx_v):
        cid = lax.axis_index("c")
        sid = lax.axis_index("s")
        wid = sid * NC + cid
        base = wid * CPT
        lane = lax.iota(jnp.int32, L)

        pltpu.sync_copy(slot_hbm, slot_v)
        pltpu.sync_copy(frame_hbm, frame_v)

        zrow = N_TOK + wid * 4 + jnp.bitwise_and(lane, 3)
        for k in range(CPT // L):
            win_v[pl.ds(k * L, L)] = zrow

        def scan(j, carry):
            s = slot_v[pl.ds(j * L, L)]
            f = frame_v[pl.ds(j * L, L)]
            c = f * N_SLOTS + s
            key = c * L + lane
            tok = j * L + lane
            ks, vs = plsc.sort_key_val(key, tok)
            cs = lax.shift_right_logical(ks, 4)
            nxt = jnp.minimum(lane + 1, L - 1)
            cs_n = lax.gather(
                cs,
                nxt[:, None],
                lax.GatherDimensionNumbers(
                    offset_dims=(),
                    collapsed_slice_dims=(0,),
                    start_index_map=(0,),
                ),
                slice_sizes=(1,),
                mode=lax.GatherScatterMode.PROMISE_IN_BOUNDS,
            )
            keep = jnp.logical_or(cs != cs_n, lane == L - 1)
            owned = jnp.logical_and(cs >= base, cs < base + CPT)
            m = jnp.logical_and(keep, owned)
            li = jnp.clip(cs - base, 0, CPT - 1)
            plsc.store_scatter(win_v, [li], vs, mask=m)
            return carry

        lax.fori_loop(0, N_TOK // L, scan, 0)

        for k in range(CPT // L):
            r, off = (k * L) // WIN, (k * L) % WIN
            idx_v[r, pl.ds(off, L)] = win_v[pl.ds(k * L, L)]
        pltpu.sync_copy(idx_v, idx_hbm.at[wid])

    return body(slot_i32, frame_i32)


def _sc_gather(idx_all, y):
    mesh = plsc.VectorSubcoreMesh(core_axis_name="c", subcore_axis_name="s")

    @functools.partial(
        pl.kernel,
        mesh=mesh,
        out_type=jax.ShapeDtypeStruct((CELLS, D), jnp.float32),
        compiler_params=pltpu.CompilerParams(
            needs_layout_passes=False, use_tc_tiling_on_sc=True
        ),
        scratch_types=[
            pltpu.VMEM((NWIN, WIN), jnp.int32),
            pltpu.VMEM((NWIN, WIN, D), jnp.float32),
        ]
        + [pltpu.SemaphoreType.DMA] * NWIN,
    )
    def body(idx_hbm, y_hbm, out_hbm, idx_v, rows_v, *sems):
        cid = lax.axis_index("c")
        sid = lax.axis_index("s")
        wid = sid * NC + cid
        base = wid * CPT

        pltpu.sync_copy(idx_hbm.at[wid], idx_v)
        cps = [
            pltpu.async_copy(y_hbm.at[idx_v.at[w]], rows_v.at[w], sems[w])
            for w in range(NWIN)
        ]
        for w in range(NWIN):
            cps[w].wait()
            pltpu.sync_copy(
                rows_v.at[w], out_hbm.at[pl.ds(base + w * WIN, WIN)]
            )

    return body(idx_all, y)


def kernel(flat_features, slot_idx, frame_ids, W1, b1, W2, b2, gamma, beta):
    slot_i32 = slot_idx.astype(jnp.int32)
    frame_i32 = frame_ids.astype(jnp.int32)
    idx_all = _sc_scan(slot_i32, frame_i32)
    y = _ffn(flat_features, W1, b1, W2, b2, gamma, beta)
    out = _sc_gather(idx_all, y)
    return out.reshape(T, N_SLOTS, D).transpose(1, 0, 2)


# speedup vs baseline: 1.7510x; 1.1536x over previous
import functools

import jax
import jax.numpy as jnp
from jax import lax
from jax.experimental import pallas as pl
from jax.experimental.pallas import tpu as pltpu
from jax.experimental.pallas import tpu_sc as plsc

N_TOK = 8192
N_SLOTS = 512
T = 30
D = 256
D_FFN = 1024
CELLS = N_SLOTS * T

L = 16
NC = 2
NS = 16
NW = NC * NS
CPT = CELLS // NW
WIN = 96
NWIN = CPT // WIN

BLK = 512
PAD = BLK
Y_ROWS = N_TOK + PAD


def _ffn_body(x_ref, w1_ref, b1_ref, w2_ref, b2_ref, g_ref, bt_ref, y_ref):
    i = pl.program_id(0)
    x = x_ref[...]
    h = jax.nn.gelu(
        jnp.dot(x, w1_ref[...], preferred_element_type=jnp.float32) + b1_ref[...]
    )
    f = jnp.dot(h, w2_ref[...], preferred_element_type=jnp.float32) + b2_ref[...]
    xr = x + f
    mean = jnp.mean(xr, axis=-1, keepdims=True)
    var = jnp.mean((xr - mean) ** 2, axis=-1, keepdims=True)
    res = (xr - mean) * lax.rsqrt(var + 1e-5) * g_ref[...] + bt_ref[...]
    y_ref[...] = jnp.where(i < N_TOK // BLK, res, 0.0)


def _ffn(x, w1, b1, w2, b2, gamma, beta):
    nblk = Y_ROWS // BLK
    last = N_TOK // BLK - 1
    return pl.pallas_call(
        _ffn_body,
        grid=(nblk,),
        in_specs=[
            pl.BlockSpec((BLK, D), lambda i: (jnp.minimum(i, last), 0)),
            pl.BlockSpec((D, D_FFN), lambda i: (0, 0)),
            pl.BlockSpec((D_FFN,), lambda i: (0,)),
            pl.BlockSpec((D_FFN, D), lambda i: (0, 0)),
            pl.BlockSpec((D,), lambda i: (0,)),
            pl.BlockSpec((D,), lambda i: (0,)),
            pl.BlockSpec((D,), lambda i: (0,)),
        ],
        out_specs=pl.BlockSpec((BLK, D), lambda i: (i, 0)),
        out_shape=jax.ShapeDtypeStruct((Y_ROWS, D), jnp.float32),
    )(x, w1, b1, w2, b2, gamma, beta)


def _sc_scan(slot_i32, frame_i32):
    mesh = plsc.VectorSubcoreMesh(core_axis_name="c", subcore_axis_name="s")

    @functools.partial(
        pl.kernel,
        mesh=mesh,
        out_type=jax.ShapeDtypeStruct((NW, NWIN, WIN), jnp.int32),
        compiler_params=pltpu.CompilerParams(
            needs_layout_passes=False, use_tc_tiling_on_sc=True
        ),
        scratch_types=[
            pltpu.VMEM((N_TOK,), jnp.int32),
            pltpu.VMEM((N_TOK,), jnp.int32),
            pltpu.VMEM((CPT,), jnp.int32),
            pltpu.VMEM((NWIN, WIN), jnp.int32),
        ],
    )
    def body(slot_hbm, frame_hbm, idx_hbm, slot_v, frame_v, win_v, idx_v):
        cid = lax.axis_index("c")
        sid = lax.axis_index("s")
        wid = sid * NC + cid
        base = wid * CPT
        lane = lax.iota(jnp.int32, L)

        pltpu.sync_copy(slot_hbm, slot_v)
        pltpu.sync_copy(frame_hbm, frame_v)

        zrow = N_TOK + wid * 4 + jnp.bitwise_and(lane, 3)
        for k in range(CPT // L):
            win_v[pl.ds(k * L, L)] = zrow

        def scan(j, carry):
            s = slot_v[pl.ds(j * L, L)]
            f = frame_v[pl.ds(j * L, L)]
            c = f * N_SLOTS + s
            key = c * L + lane
            tok = j * L + lane
            ks, vs = plsc.sort_key_val(key, tok)
            cs = lax.shift_right_logical(ks, 4)
            nxt = jnp.minimum(lane + 1, L - 1)
            cs_n = lax.gather(
                cs,
                nxt[:, None],
                lax.GatherDimensionNumbers(
                    offset_dims=(),
                    collapsed_slice_dims=(0,),
                    start_index_map=(0,),
                ),
                slice_sizes=(1,),
                mode=lax.GatherScatterMode.PROMISE_IN_BOUNDS,
            )
            keep = jnp.logical_or(cs != cs_n, lane == L - 1)
            owned = jnp.logical_and(cs >= base, cs < base + CPT)
            m = jnp.logical_and(keep, owned)
            li = jnp.clip(cs - base, 0, CPT - 1)
            plsc.store_scatter(win_v, [li], vs, mask=m)
            return carry

        lax.fori_loop(0, N_TOK // L, scan, 0)

        for k in range(CPT // L):
            r, off = (k * L) // WIN, (k * L) % WIN
            idx_v[r, pl.ds(off, L)] = win_v[pl.ds(k * L, L)]
        pltpu.sync_copy(idx_v, idx_hbm.at[wid])

    return body(slot_i32, frame_i32)


def _sc_gather(idx_all, y):
    mesh = plsc.VectorSubcoreMesh(core_axis_name="c", subcore_axis_name="s")

    @functools.partial(
        pl.kernel,
        mesh=mesh,
        out_type=jax.ShapeDtypeStruct((CELLS, D), jnp.float32),
        compiler_params=pltpu.CompilerParams(
            needs_layout_passes=False, use_tc_tiling_on_sc=True
        ),
        scratch_types=[
            pltpu.VMEM((NWIN, WIN), jnp.int32),
            pltpu.VMEM((NWIN, WIN, D), jnp.float32),
        ]
        + [pltpu.SemaphoreType.DMA] * (2 * NWIN),
    )
    def body(idx_hbm, y_hbm, out_hbm, idx_v, rows_v, *sems):
        cid = lax.axis_index("c")
        sid = lax.axis_index("s")
        wid = sid * NC + cid
        base = wid * CPT

        pltpu.sync_copy(idx_hbm.at[wid], idx_v)
        cps = [
            pltpu.async_copy(y_hbm.at[idx_v.at[w]], rows_v.at[w], sems[w])
            for w in range(NWIN)
        ]
        wps = []
        for w in range(NWIN):
            cps[w].wait()
            wps.append(
                pltpu.async_copy(
                    rows_v.at[w],
                    out_hbm.at[pl.ds(base + w * WIN, WIN)],
                    sems[NWIN + w],
                )
            )
        for w in range(NWIN):
            wps[w].wait()

    return body(idx_all, y)


def kernel(flat_features, slot_idx, frame_ids, W1, b1, W2, b2, gamma, beta):
    slot_i32 = slot_idx.astype(jnp.int32)
    frame_i32 = frame_ids.astype(jnp.int32)
    idx_all = _sc_scan(slot_i32, frame_i32)
    y = _ffn(flat_features, W1, b1, W2, b2, gamma, beta)
    out = _sc_gather(idx_all, y)
    return out.reshape(T, N_SLOTS, D).transpose(1, 0, 2)


# speedup vs baseline: 1.9925x; 1.1379x over previous
import functools

import jax
import jax.numpy as jnp
from jax import lax
from jax.experimental import pallas as pl
from jax.experimental.pallas import tpu as pltpu
from jax.experimental.pallas import tpu_sc as plsc

N_TOK = 8192
N_SLOTS = 512
T = 30
D = 256
D_FFN = 1024
CELLS = N_SLOTS * T

L = 16
NC = 2
NS = 16
NW = NC * NS
CPT = CELLS // NW
WIN = 96
NWIN = CPT // WIN

BLK = 512
PAD = BLK
Y_ROWS = N_TOK + PAD


def _ffn_body(x_ref, w1_ref, b1_ref, w2_ref, b2_ref, g_ref, bt_ref, y_ref):
    i = pl.program_id(0)
    x = x_ref[...]
    h = jax.nn.gelu(
        jnp.dot(x, w1_ref[...], preferred_element_type=jnp.float32) + b1_ref[...]
    )
    f = jnp.dot(h, w2_ref[...], preferred_element_type=jnp.float32) + b2_ref[...]
    xr = x + f
    mean = jnp.mean(xr, axis=-1, keepdims=True)
    var = jnp.mean((xr - mean) ** 2, axis=-1, keepdims=True)
    res = (xr - mean) * lax.rsqrt(var + 1e-5) * g_ref[...] + bt_ref[...]
    y_ref[...] = jnp.where(i < N_TOK // BLK, res, 0.0)


def _ffn(x, w1, b1, w2, b2, gamma, beta):
    nblk = Y_ROWS // BLK
    last = N_TOK // BLK - 1
    return pl.pallas_call(
        _ffn_body,
        grid=(nblk,),
        in_specs=[
            pl.BlockSpec((BLK, D), lambda i: (jnp.minimum(i, last), 0)),
            pl.BlockSpec((D, D_FFN), lambda i: (0, 0)),
            pl.BlockSpec((D_FFN,), lambda i: (0,)),
            pl.BlockSpec((D_FFN, D), lambda i: (0, 0)),
            pl.BlockSpec((D,), lambda i: (0,)),
            pl.BlockSpec((D,), lambda i: (0,)),
            pl.BlockSpec((D,), lambda i: (0,)),
        ],
        out_specs=pl.BlockSpec((BLK, D), lambda i: (i, 0)),
        out_shape=jax.ShapeDtypeStruct((Y_ROWS, D), jnp.float32),
    )(x, w1, b1, w2, b2, gamma, beta)


def _sc_scan(slot_i32, frame_i32):
    mesh = plsc.VectorSubcoreMesh(core_axis_name="c", subcore_axis_name="s")

    @functools.partial(
        pl.kernel,
        mesh=mesh,
        out_type=jax.ShapeDtypeStruct((NW, NWIN, WIN), jnp.int32),
        compiler_params=pltpu.CompilerParams(
            needs_layout_passes=False, use_tc_tiling_on_sc=True
        ),
        scratch_types=[
            pltpu.VMEM((N_TOK,), jnp.int32),
            pltpu.VMEM((N_TOK,), jnp.int32),
            pltpu.VMEM((CPT,), jnp.int32),
            pltpu.VMEM((NWIN, WIN), jnp.int32),
        ],
    )
    def body(slot_hbm, frame_hbm, idx_hbm, slot_v, frame_v, win_v, idx_v):
        cid = lax.axis_index("c")
        sid = lax.axis_index("s")
        wid = sid * NC + cid
        base = wid * CPT
        lane = lax.iota(jnp.int32, L)

        pltpu.sync_copy(slot_hbm, slot_v)
        pltpu.sync_copy(frame_hbm, frame_v)

        for k in range(CPT // L):
            win_v[pl.ds(k * L, L)] = N_TOK + k * L + lane

        def scan(j, carry):
            s = slot_v[pl.ds(j * L, L)]
            f = frame_v[pl.ds(j * L, L)]
            c = f * N_SLOTS + s
            key = c * L + lane
            tok = j * L + lane
            ks, vs = plsc.sort_key_val(key, tok)
            cs = lax.shift_right_logical(ks, 4)
            nxt = jnp.minimum(lane + 1, L - 1)
            cs_n = lax.gather(
                cs,
                nxt[:, None],
                lax.GatherDimensionNumbers(
                    offset_dims=(),
                    collapsed_slice_dims=(0,),
                    start_index_map=(0,),
                ),
                slice_sizes=(1,),
                mode=lax.GatherScatterMode.PROMISE_IN_BOUNDS,
            )
            keep = jnp.logical_or(cs != cs_n, lane == L - 1)
            owned = jnp.logical_and(cs >= base, cs < base + CPT)
            m = jnp.logical_and(keep, owned)
            li = jnp.clip(cs - base, 0, CPT - 1)
            plsc.store_scatter(win_v, [li], vs, mask=m)
            return carry

        lax.fori_loop(0, N_TOK // L, scan, 0)

        for k in range(CPT // L):
            r, off = (k * L) // WIN, (k * L) % WIN
            idx_v[r, pl.ds(off, L)] = win_v[pl.ds(k * L, L)]
        pltpu.sync_copy(idx_v, idx_hbm.at[wid])

    return body(slot_i32, frame_i32)


def _sc_gather(idx_all, y):
    mesh = plsc.VectorSubcoreMesh(core_axis_name="c", subcore_axis_name="s")

    @functools.partial(
        pl.kernel,
        mesh=mesh,
        out_type=jax.ShapeDtypeStruct((CELLS, D), jnp.float32),
        compiler_params=pltpu.CompilerParams(
            needs_layout_passes=False, use_tc_tiling_on_sc=True
        ),
        scratch_types=[
            pltpu.VMEM((NWIN, WIN), jnp.int32),
            pltpu.VMEM((NWIN, WIN, D), jnp.float32),
        ]
        + [pltpu.SemaphoreType.DMA] * (2 * NWIN),
    )
    def body(idx_hbm, y_hbm, out_hbm, idx_v, rows_v, *sems):
        cid = lax.axis_index("c")
        sid = lax.axis_index("s")
        wid = sid * NC + cid
        base = wid * CPT

        pltpu.sync_copy(idx_hbm.at[wid], idx_v)
        cps = [
            pltpu.async_copy(y_hbm.at[idx_v.at[w]], rows_v.at[w], sems[w])
            for w in range(NWIN)
        ]
        wps = []
        for w in range(NWIN):
            cps[w].wait()
            wps.append(
                pltpu.async_copy(
                    rows_v.at[w],
                    out_hbm.at[pl.ds(base + w * WIN, WIN)],
                    sems[NWIN + w],
                )
            )
        for w in range(NWIN):
            wps[w].wait()

    return body(idx_all, y)


def kernel(flat_features, slot_idx, frame_ids, W1, b1, W2, b2, gamma, beta):
    slot_i32 = slot_idx.astype(jnp.int32)
    frame_i32 = frame_ids.astype(jnp.int32)
    idx_all = _sc_scan(slot_i32, frame_i32)
    y = _ffn(flat_features, W1, b1, W2, b2, gamma, beta)
    out = _sc_gather(idx_all, y)
    return out.reshape(T, N_SLOTS, D).transpose(1, 0, 2)


# speedup vs baseline: 2.0743x; 1.0410x over previous
import functools

import jax
import jax.numpy as jnp
from jax import lax
from jax.experimental import pallas as pl
from jax.experimental.pallas import tpu as pltpu
from jax.experimental.pallas import tpu_sc as plsc

N_TOK = 8192
N_SLOTS = 512
T = 30
D = 256
D_FFN = 1024
CELLS = N_SLOTS * T

L = 16
NC = 2
NS = 16
NW = NC * NS
CPT = CELLS // NW
WIN = 96
NWIN = CPT // WIN

BLK = 1024
PAD = BLK
Y_ROWS = N_TOK + PAD


def _ffn_body(x_ref, w1_ref, b1_ref, w2_ref, b2_ref, g_ref, bt_ref, y_ref):
    i = pl.program_id(0)
    x = x_ref[...]
    h = jax.nn.gelu(
        jnp.dot(x, w1_ref[...], preferred_element_type=jnp.float32) + b1_ref[...]
    )
    f = jnp.dot(h, w2_ref[...], preferred_element_type=jnp.float32) + b2_ref[...]
    xr = x + f
    mean = jnp.mean(xr, axis=-1, keepdims=True)
    var = jnp.mean((xr - mean) ** 2, axis=-1, keepdims=True)
    res = (xr - mean) * lax.rsqrt(var + 1e-5) * g_ref[...] + bt_ref[...]
    y_ref[...] = jnp.where(i < N_TOK // BLK, res, 0.0)


def _ffn(x, w1, b1, w2, b2, gamma, beta):
    nblk = Y_ROWS // BLK
    last = N_TOK // BLK - 1
    return pl.pallas_call(
        _ffn_body,
        grid=(nblk,),
        in_specs=[
            pl.BlockSpec((BLK, D), lambda i: (jnp.minimum(i, last), 0)),
            pl.BlockSpec((D, D_FFN), lambda i: (0, 0)),
            pl.BlockSpec((D_FFN,), lambda i: (0,)),
            pl.BlockSpec((D_FFN, D), lambda i: (0, 0)),
            pl.BlockSpec((D,), lambda i: (0,)),
            pl.BlockSpec((D,), lambda i: (0,)),
            pl.BlockSpec((D,), lambda i: (0,)),
        ],
        out_specs=pl.BlockSpec((BLK, D), lambda i: (i, 0)),
        out_shape=jax.ShapeDtypeStruct((Y_ROWS, D), jnp.float32),
    )(x, w1, b1, w2, b2, gamma, beta)


def _sc_scan(slot_i32, frame_i32):
    mesh = plsc.VectorSubcoreMesh(core_axis_name="c", subcore_axis_name="s")

    @functools.partial(
        pl.kernel,
        mesh=mesh,
        out_type=jax.ShapeDtypeStruct((NW, NWIN, WIN), jnp.int32),
        compiler_params=pltpu.CompilerParams(
            needs_layout_passes=False, use_tc_tiling_on_sc=True
        ),
        scratch_types=[
            pltpu.VMEM((N_TOK,), jnp.int32),
            pltpu.VMEM((N_TOK,), jnp.int32),
            pltpu.VMEM((CPT,), jnp.int32),
            pltpu.VMEM((NWIN, WIN), jnp.int32),
        ],
    )
    def body(slot_hbm, frame_hbm, idx_hbm, slot_v, frame_v, win_v, idx_v):
        cid = lax.axis_index("c")
        sid = lax.axis_index("s")
        wid = sid * NC + cid
        base = wid * CPT
        lane = lax.iota(jnp.int32, L)

        pltpu.sync_copy(slot_hbm, slot_v)
        pltpu.sync_copy(frame_hbm, frame_v)

        for k in range(CPT // L):
            win_v[pl.ds(k * L, L)] = N_TOK + k * L + lane

        def scan(j, carry):
            s = slot_v[pl.ds(j * L, L)]
            f = frame_v[pl.ds(j * L, L)]
            c = f * N_SLOTS + s
            key = c * L + lane
            tok = j * L + lane
            ks, vs = plsc.sort_key_val(key, tok)
            cs = lax.shift_right_logical(ks, 4)
            nxt = jnp.minimum(lane + 1, L - 1)
            cs_n = lax.gather(
                cs,
                nxt[:, None],
                lax.GatherDimensionNumbers(
                    offset_dims=(),
                    collapsed_slice_dims=(0,),
                    start_index_map=(0,),
                ),
                slice_sizes=(1,),
                mode=lax.GatherScatterMode.PROMISE_IN_BOUNDS,
            )
            keep = jnp.logical_or(cs != cs_n, lane == L - 1)
            owned = jnp.logical_and(cs >= base, cs < base + CPT)
            m = jnp.logical_and(keep, owned)
            li = jnp.clip(cs - base, 0, CPT - 1)
            plsc.store_scatter(win_v, [li], vs, mask=m)
            return carry

        lax.fori_loop(0, N_TOK // L, scan, 0)

        for k in range(CPT // L):
            r, off = (k * L) // WIN, (k * L) % WIN
            idx_v[r, pl.ds(off, L)] = win_v[pl.ds(k * L, L)]
        pltpu.sync_copy(idx_v, idx_hbm.at[wid])

    return body(slot_i32, frame_i32)


def _sc_gather(idx_all, y):
    mesh = plsc.VectorSubcoreMesh(core_axis_name="c", subcore_axis_name="s")

    @functools.partial(
        pl.kernel,
        mesh=mesh,
        out_type=jax.ShapeDtypeStruct((CELLS, D), jnp.float32),
        compiler_params=pltpu.CompilerParams(
            needs_layout_passes=False, use_tc_tiling_on_sc=True
        ),
        scratch_types=[
            pltpu.VMEM((NWIN, WIN), jnp.int32),
            pltpu.VMEM((NWIN, WIN, D), jnp.float32),
        ]
        + [pltpu.SemaphoreType.DMA] * (2 * NWIN),
    )
    def body(idx_hbm, y_hbm, out_hbm, idx_v, rows_v, *sems):
        cid = lax.axis_index("c")
        sid = lax.axis_index("s")
        wid = sid * NC + cid
        base = wid * CPT

        pltpu.sync_copy(idx_hbm.at[wid], idx_v)
        cps = [
            pltpu.async_copy(y_hbm.at[idx_v.at[w]], rows_v.at[w], sems[w])
            for w in range(NWIN)
        ]
        wps = []
        for w in range(NWIN):
            cps[w].wait()
            wps.append(
                pltpu.async_copy(
                    rows_v.at[w],
                    out_hbm.at[pl.ds(base + w * WIN, WIN)],
                    sems[NWIN + w],
                )
            )
        for w in range(NWIN):
            wps[w].wait()

    return body(idx_all, y)


def kernel(flat_features, slot_idx, frame_ids, W1, b1, W2, b2, gamma, beta):
    slot_i32 = slot_idx.astype(jnp.int32)
    frame_i32 = frame_ids.astype(jnp.int32)
    idx_all = _sc_scan(slot_i32, frame_i32)
    y = _ffn(flat_features, W1, b1, W2, b2, gamma, beta)
    out = _sc_gather(idx_all, y)
    return out.reshape(T, N_SLOTS, D).transpose(1, 0, 2)


# speedup vs baseline: 2.1553x; 1.0390x over previous
import functools

import jax
import jax.numpy as jnp
from jax import lax
from jax.experimental import pallas as pl
from jax.experimental.pallas import tpu as pltpu
from jax.experimental.pallas import tpu_sc as plsc

N_TOK = 8192
N_SLOTS = 512
T = 30
D = 256
D_FFN = 1024
CELLS = N_SLOTS * T

L = 16
NC = 2
NS = 16
NW = NC * NS
CPT = CELLS // NW
WIN = 96
NWIN = CPT // WIN

BLK = 1024
PAD = BLK
Y_ROWS = N_TOK + PAD


def _ffn_body(x_ref, w1_ref, b1_ref, w2_ref, b2_ref, g_ref, bt_ref, y_ref):
    i = pl.program_id(0)

    @pl.when(i < N_TOK // BLK)
    def _compute():
        x = x_ref[...]
        z = jnp.dot(x, w1_ref[...], preferred_element_type=jnp.float32) + b1_ref[...]
        zu = z * (-1.5957691216057308 + (-0.07135481262980924) * (z * z))
        h = z / (1.0 + jnp.exp(zu))
        f = jnp.dot(h, w2_ref[...], preferred_element_type=jnp.float32) + b2_ref[...]
        xr = x + f
        mean = jnp.mean(xr, axis=-1, keepdims=True)
        var = jnp.mean((xr - mean) ** 2, axis=-1, keepdims=True)
        y_ref[...] = (xr - mean) * lax.rsqrt(var + 1e-5) * g_ref[...] + bt_ref[...]

    @pl.when(i >= N_TOK // BLK)
    def _pad():
        y_ref[...] = jnp.zeros_like(y_ref)


def _ffn(x, w1, b1, w2, b2, gamma, beta):
    nblk = Y_ROWS // BLK
    last = N_TOK // BLK - 1
    return pl.pallas_call(
        _ffn_body,
        grid=(nblk,),
        in_specs=[
            pl.BlockSpec((BLK, D), lambda i: (jnp.minimum(i, last), 0)),
            pl.BlockSpec((D, D_FFN), lambda i: (0, 0)),
            pl.BlockSpec((D_FFN,), lambda i: (0,)),
            pl.BlockSpec((D_FFN, D), lambda i: (0, 0)),
            pl.BlockSpec((D,), lambda i: (0,)),
            pl.BlockSpec((D,), lambda i: (0,)),
            pl.BlockSpec((D,), lambda i: (0,)),
        ],
        out_specs=pl.BlockSpec((BLK, D), lambda i: (i, 0)),
        out_shape=jax.ShapeDtypeStruct((Y_ROWS, D), jnp.float32),
    )(x, w1, b1, w2, b2, gamma, beta)


def _sc_scan(slot_i32, frame_i32):
    mesh = plsc.VectorSubcoreMesh(core_axis_name="c", subcore_axis_name="s")

    @functools.partial(
        pl.kernel,
        mesh=mesh,
        out_type=jax.ShapeDtypeStruct((NW, NWIN, WIN), jnp.int32),
        compiler_params=pltpu.CompilerParams(
            needs_layout_passes=False, use_tc_tiling_on_sc=True
        ),
        scratch_types=[
            pltpu.VMEM((N_TOK,), jnp.int32),
            pltpu.VMEM((N_TOK,), jnp.int32),
            pltpu.VMEM((CPT,), jnp.int32),
            pltpu.VMEM((NWIN, WIN), jnp.int32),
        ],
    )
    def body(slot_hbm, frame_hbm, idx_hbm, slot_v, frame_v, win_v, idx_v):
        cid = lax.axis_index("c")
        sid = lax.axis_index("s")
        wid = sid * NC + cid
        base = wid * CPT
        lane = lax.iota(jnp.int32, L)

        pltpu.sync_copy(slot_hbm, slot_v)
        pltpu.sync_copy(frame_hbm, frame_v)

        for k in range(CPT // L):
            win_v[pl.ds(k * L, L)] = N_TOK + k * L + lane

        def scan(j, carry):
            s = slot_v[pl.ds(j * L, L)]
            f = frame_v[pl.ds(j * L, L)]
            c = f * N_SLOTS + s
            key = c * L + lane
            tok = j * L + lane
            ks, vs = plsc.sort_key_val(key, tok)
            cs = lax.shift_right_logical(ks, 4)
            nxt = jnp.minimum(lane + 1, L - 1)
            cs_n = lax.gather(
                cs,
                nxt[:, None],
                lax.GatherDimensionNumbers(
                    offset_dims=(),
                    collapsed_slice_dims=(0,),
                    start_index_map=(0,),
                ),
                slice_sizes=(1,),
                mode=lax.GatherScatterMode.PROMISE_IN_BOUNDS,
            )
            keep = jnp.logical_or(cs != cs_n, lane == L - 1)
            owned = jnp.logical_and(cs >= base, cs < base + CPT)
            m = jnp.logical_and(keep, owned)
            li = jnp.clip(cs - base, 0, CPT - 1)
            plsc.store_scatter(win_v, [li], vs, mask=m)
            return carry

        lax.fori_loop(0, N_TOK // L, scan, 0)

        for k in range(CPT // L):
            r, off = (k * L) // WIN, (k * L) % WIN
            idx_v[r, pl.ds(off, L)] = win_v[pl.ds(k * L, L)]
        pltpu.sync_copy(idx_v, idx_hbm.at[wid])

    return body(slot_i32, frame_i32)


def _sc_gather(idx_all, y):
    mesh = plsc.VectorSubcoreMesh(core_axis_name="c", subcore_axis_name="s")

    @functools.partial(
        pl.kernel,
        mesh=mesh,
        out_type=jax.ShapeDtypeStruct((CELLS, D), jnp.float32),
        compiler_params=pltpu.CompilerParams(
            needs_layout_passes=False, use_tc_tiling_on_sc=True
        ),
        scratch_types=[
            pltpu.VMEM((NWIN, WIN), jnp.int32),
            pltpu.VMEM((NWIN, WIN, D), jnp.float32),
        ]
        + [pltpu.SemaphoreType.DMA] * (2 * NWIN),
    )
    def body(idx_hbm, y_hbm, out_hbm, idx_v, rows_v, *sems):
        cid = lax.axis_index("c")
        sid = lax.axis_index("s")
        wid = sid * NC + cid
        base = wid * CPT

        pltpu.sync_copy(idx_hbm.at[wid], idx_v)
        cps = [
            pltpu.async_copy(y_hbm.at[idx_v.at[w]], rows_v.at[w], sems[w])
            for w in range(NWIN)
        ]
        wps = []
        for w in range(NWIN):
            cps[w].wait()
            wps.append(
                pltpu.async_copy(
                    rows_v.at[w],
                    out_hbm.at[pl.ds(base + w * WIN, WIN)],
                    sems[NWIN + w],
                )
            )
        for w in range(NWIN):
            wps[w].wait()

    return body(idx_all, y)


def kernel(flat_features, slot_idx, frame_ids, W1, b1, W2, b2, gamma, beta):
    slot_i32 = slot_idx.astype(jnp.int32)
    frame_i32 = frame_ids.astype(jnp.int32)
    idx_all = _sc_scan(slot_i32, frame_i32)
    y = _ffn(flat_features, W1, b1, W2, b2, gamma, beta)
    out = _sc_gather(idx_all, y)
    return out.reshape(T, N_SLOTS, D).transpose(1, 0, 2)
